# baseline (speedup 1.0000x reference)
import functools

import jax
import jax.numpy as jnp
from jax import lax
from jax.experimental import pallas as pl
from jax.experimental.pallas import tpu as pltpu
from jax.experimental.pallas import tpu_sc as plsc

N_RAW = 10000
N_PAD = 10240
BN = 1280
NB = N_PAD // BN
NCFG_PAD = 1024
E_RAW = 320000
E_PAD = 327680
EROWS = 2560
TROWS = EROWS // 16

_SC_MESH = plsc.VectorSubcoreMesh(core_axis_name="c", subcore_axis_name="s",
                                  num_cores=2, num_subcores=16)
_SC_PARAMS = pltpu.CompilerParams(needs_layout_passes=False,
                                  use_tc_tiling_on_sc=False)


def _i16(val):
    return jnp.zeros((16,), jnp.int32) + val


def _sc_pre_body(dst_ref, opc_ref, emb_ref, ids_ref, cfg_ref,
                 cnt_out, op16_out, cfg_out,
                 dstv, cntw, opcv, oprows, idsv, cfgv, sem):
    c = lax.axis_index("c")
    s = lax.axis_index("s")

    @pl.when(c == 0)
    def _():
        zero16 = jnp.zeros((16,), jnp.float32)

        def zrow(i, carry):
            cntw[pl.ds(i * 16, 16)] = zero16
            return carry
        lax.fori_loop(0, N_PAD // 16, zrow, 0)

        pltpu.sync_copy(dst_ref.at[pl.ds(s * TROWS, TROWS)], dstv)
        ones = jnp.zeros((16,), jnp.float32) + 1.0

        def count(i, carry):
            r = i // 8
            k = i % 8
            d16 = dstv[r, pl.ds(k * 16, 16)]
            plsc.addupdate_scatter(cntw, [d16], ones)
            return carry
        lax.fori_loop(0, TROWS * 8, count, 0)
        pltpu.sync_copy(cntw, cnt_out.at[s])

    @pl.when(c == 1)
    def _():
        pltpu.sync_copy(opc_ref.at[s], opcv)
        for j in range(5):
            pltpu.async_copy(emb_ref.at[opcv.at[j]], oprows, sem).wait()
            pltpu.sync_copy(oprows, op16_out.at[pl.ds(s * 640 + j * 128, 128)])

        pltpu.sync_copy(ids_ref, idsv)
        pltpu.sync_copy(cfg_ref.at[pl.ds(s * 64, 64)], cfgv)
        iota16 = jnp.arange(16, dtype=jnp.int32)

        def cbody(j, carry):
            cc = s * 64 + j
            myid = plsc.load_gather(idsv, [_i16(cc)])

            def chunk(k, acc):
                vals = idsv[pl.ds(k * 16, 16)]
                pos = iota16 + k * 16
                hit = jnp.where((vals == myid) & (pos > cc), 1.0, 0.0)
                return jnp.maximum(acc, hit)
            acc = lax.fori_loop(0, 64, chunk, jnp.zeros((16,), jnp.float32))
            keep = jnp.where(jnp.max(acc) > 0.0, 0.0, 1.0)
            cfgv[j, pl.ds(0, 16)] = cfgv[j, pl.ds(0, 16)] * keep
            cfgv[j, pl.ds(16, 16)] = cfgv[j, pl.ds(16, 16)] * keep
            return carry
        lax.fori_loop(0, 64, cbody, 0)
        pltpu.sync_copy(cfgv, cfg_out.at[pl.ds(s * 64, 64)])


def _sc_pre(dstA, opc2, embp, ids_i, cfg32):
    f32 = jnp.float32
    return pl.kernel(
        _sc_pre_body,
        out_type=[
            jax.ShapeDtypeStruct((16, N_PAD), f32),
            jax.ShapeDtypeStruct((N_PAD, 16), f32),
            jax.ShapeDtypeStruct((NCFG_PAD, 32), f32)
        ],
        mesh=_SC_MESH,
        scratch_types=[
            pltpu.VMEM((TROWS, 128), jnp.int32),
            pltpu.VMEM((N_PAD,), f32),
            pltpu.VMEM((5, 128), jnp.int32),
            pltpu.VMEM((128, 16), f32),
            pltpu.VMEM((NCFG_PAD,), jnp.int32),
            pltpu.VMEM((64, 32), f32),
            pltpu.SemaphoreType.DMA,
        ],
        compiler_params=_SC_PARAMS,
    )(dstA, opc2, embp, ids_i, cfg32)


def _spmv_body(x_ref, src0_ref, src1_ref, dst_ref, inv_ref, out_ref,
               sidx, didx, wb, invv, acc_sh, gsem, ssem, *bufs, ch, nbuf):
    c = lax.axis_index("c")
    s = lax.axis_index("s")
    zero16 = jnp.zeros((16,), jnp.float32)

    def zrow(i, carry):
        for m in range(ch // 16):
            wb[i, pl.ds(m * 16, 16)] = zero16
        return carry
    lax.fori_loop(0, 64, zrow, 0)

    def zslab(t, carry):
        pltpu.sync_copy(wb, acc_sh.at[pl.ds(s * 640 + t * 64, 64)])
        return carry
    lax.fori_loop(0, 10, zslab, 0)
    plsc.subcore_barrier()

    @pl.when(c == 0)
    def _():
        pltpu.sync_copy(src0_ref.at[pl.ds(s * TROWS, TROWS)], sidx)

    @pl.when(c == 1)
    def _():
        pltpu.sync_copy(src1_ref.at[pl.ds(s * TROWS, TROWS)], sidx)
    pltpu.sync_copy(dst_ref.at[pl.ds(s * TROWS, TROWS)], didx)

    dep = max(1, nbuf // 4)

    def fire_g(b, t):
        pltpu.async_copy(x_ref.at[sidx.at[t]], bufs[b], gsem.at[b])

    def fire_s(b, u):
        pltpu.async_copy(bufs[b], acc_sh.at[didx.at[u]], ssem.at[b], add=True)

    def wait_g(b):
        pltpu.make_async_copy(x_ref.at[sidx.at[0]], bufs[b], gsem.at[b]).wait()

    def wait_s(b):
        pltpu.make_async_copy(bufs[b], acc_sh.at[didx.at[0]],
                              ssem.at[b]).wait()

    for j in range(16):
        b = j % nbuf
        if j >= nbuf:
            wait_s(b)
        fire_g(b, j)
        if j >= dep:
            wait_g((j - dep) % nbuf)
            fire_s((j - dep) % nbuf, j - dep)

    def sup(q, carry):
        t0 = q * 16
        for j in range(16):
            b = j % nbuf
            wait_s(b)
            fire_g(b, t0 + j)
            wait_g((j - dep) % nbuf)
            fire_s((j - dep) % nbuf, t0 + j - dep)
        return carry
    lax.fori_loop(1, 10, sup, 0)

    for u in range(TROWS - dep, TROWS):
        wait_g(u % nbuf)
        fire_s(u % nbuf, u)
    for b in range(nbuf):
        wait_s(b)
    plsc.subcore_barrier()

    pltpu.sync_copy(inv_ref.at[pl.ds(s * 640, 640)], invv)

    def wo(t, carry):
        r0 = s * 640 + t * 64
        pltpu.sync_copy(acc_sh.at[pl.ds(r0, 64)], wb)

        def row(r, carry2):
            iv = plsc.load_gather(invv, [_i16(t * 64 + r)])
            for m in range(ch // 16):
                wb[r, pl.ds(m * 16, 16)] = wb[r, pl.ds(m * 16, 16)] * iv
            return carry2
        lax.fori_loop(0, 64, row, 0)
        pltpu.sync_copy(wb, out_ref.at[pl.ds(c * N_PAD + r0, 64)])
        return carry
    lax.fori_loop(0, 10, wo, 0)


@functools.lru_cache(maxsize=None)
def _sc_spmv(ch):
    f32 = jnp.float32
    nbuf = 8 if ch <= 32 else 4
    return pl.kernel(
        functools.partial(_spmv_body, ch=ch, nbuf=nbuf),
        out_type=jax.ShapeDtypeStruct((2 * N_PAD, ch), f32),
        mesh=_SC_MESH,
        scratch_types=[
            pltpu.VMEM((TROWS, 128), jnp.int32),
            pltpu.VMEM((TROWS, 128), jnp.int32),
            pltpu.VMEM((64, ch), f32),
            pltpu.VMEM((640,), f32),
            pltpu.VMEM_SHARED((N_PAD, ch), f32),
            pltpu.SemaphoreType.DMA((nbuf,)),
            pltpu.SemaphoreType.DMA((nbuf,)),
        ] + [pltpu.VMEM((128, ch), f32)] * nbuf,
        compiler_params=_SC_PARAMS,
    )


def _pre_body(nf_ref, op16_ref, cfg_eff_ref, ids_ref, cnt_ref, wf_ref, bf_ref,
              wi1_ref, wi2_ref, wi3_ref, bi_ref, out_ref, inv_ref):
    cnt = jnp.sum(cnt_ref[...], axis=0, keepdims=True)
    inv_ref[0] = 1.0 / jnp.maximum(cnt, 1.0)
    feat = jnp.log1p(jax.nn.relu(nf_ref[...]))
    femb = jnp.dot(feat, wf_ref[...], preferred_element_type=jnp.float32) + bf_ref[...]
    t1 = jnp.dot(femb, wi1_ref[...], preferred_element_type=jnp.float32)
    t2 = jnp.dot(op16_ref[...], wi2_ref[...], preferred_element_type=jnp.float32)
    u = jnp.dot(cfg_eff_ref[...], wi3_ref[...], preferred_element_type=jnp.float32)
    ni = (pl.program_id(0) * BN
          + lax.broadcasted_iota(jnp.int32, (BN, 1), 0)).astype(jnp.float32)
    oh = (ni == ids_ref[...]).astype(jnp.float32)
    inj = jnp.dot(oh, u, preferred_element_type=jnp.float32)
    y = jax.nn.relu(t1 + t2 + inj + bi_ref[...])
    out_ref[0] = y[:, :16]
    out_ref[1] = y[:, 16:]


def _tc_pre(nf_pad, op16, cfg_eff, ids_f, cntparts, W_feat, b_feat,
            Wi1, Wi2, Wi3, b_in):
    return pl.pallas_call(
        _pre_body,
        grid=(NB,),
        in_specs=[
            pl.BlockSpec((BN, 140), lambda i: (i, 0)),
            pl.BlockSpec((BN, 16), lambda i: (i, 0)),
            pl.BlockSpec((NCFG_PAD, 32), lambda i: (0, 0)),
            pl.BlockSpec((1, NCFG_PAD), lambda i: (0, 0)),
            pl.BlockSpec((16, BN), lambda i: (0, i)),
            pl.BlockSpec((140, 20), lambda i: (0, 0)),
            pl.BlockSpec((1, 20), lambda i: (0, 0)),
            pl.BlockSpec((20, 32), lambda i: (0, 0)),
            pl.BlockSpec((16, 32), lambda i: (0, 0)),
            pl.BlockSpec((32, 32), lambda i: (0, 0)),
            pl.BlockSpec((1, 32), lambda i: (0, 0)),
        ],
        out_specs=[
            pl.BlockSpec((2, BN, 16), lambda i: (0, i, 0)),
            pl.BlockSpec((1, 1, BN), lambda i: (i, 0, 0)),
        ],
        out_shape=[
            jax.ShapeDtypeStruct((2, N_PAD, 16), jnp.float32),
            jax.ShapeDtypeStruct((NB, 1, BN), jnp.float32),
        ],
    )(nf_pad, op16, cfg_eff, ids_f, cntparts, W_feat, b_feat,
      Wi1, Wi2, Wi3, b_in)


def _layer_body(x_ref, m_ref, wl_ref, wr_ref, bl_ref, out_ref, *, ki, ko, co):
    xc = jnp.concatenate([x_ref[q] for q in range(ki)], axis=-1)
    mc = jnp.concatenate([m_ref[q] for q in range(ki)], axis=-1)
    y = jnp.dot(mc, wl_ref[...], preferred_element_type=jnp.float32)
    y = y + jnp.dot(xc, wr_ref[...], preferred_element_type=jnp.float32)
    y = jax.nn.relu(y + bl_ref[...])
    cq = co // ko
    for q in range(ko):
        out_ref[q] = y[:, q * cq:(q + 1) * cq]


@functools.lru_cache(maxsize=None)
def _tc_layer(ci, co, ki, ko):
    ciq, coq = ci // ki, co // ko
    return pl.pallas_call(
        functools.partial(_layer_body, ki=ki, ko=ko, co=co),
        grid=(NB,),
        in_specs=[
            pl.BlockSpec((ki, BN, ciq), lambda i: (0, i, 0)),
            pl.BlockSpec((ki, BN, ciq), lambda i: (0, i, 0)),
            pl.BlockSpec((ci, co), lambda i: (0, 0)),
            pl.BlockSpec((ci, co), lambda i: (0, 0)),
            pl.BlockSpec((1, co), lambda i: (0, 0)),
        ],
        out_specs=pl.BlockSpec((ko, BN, coq), lambda i: (0, i, 0)),
        out_shape=jax.ShapeDtypeStruct((ko, N_PAD, coq), jnp.float32),
    )


def _final_body(x_ref, wout_ref, bout_ref, batch_ref, out_ref):
    xc = jnp.concatenate([x_ref[q] for q in range(4)], axis=-1)
    v = jnp.dot(xc, wout_ref[...], preferred_element_type=jnp.float32) + bout_ref[...]
    gr = lax.broadcasted_iota(jnp.int32, (1, 16), 1).astype(jnp.float32)
    oh = (batch_ref[...] == gr).astype(jnp.float32)
    po = 0.001 * lax.dot_general(oh, v, (((0,), (0,)), ((), ())),
                                 preferred_element_type=jnp.float32)

    @pl.when(pl.program_id(0) == 0)
    def _():
        out_ref[...] = po

    @pl.when(pl.program_id(0) != 0)
    def _():
        out_ref[...] += po


def _tc_final(x2, W_out, b_out, batch_col):
    return pl.pallas_call(
        _final_body,
        grid=(NB,),
        in_specs=[
            pl.BlockSpec((4, BN, 64), lambda i: (0, i, 0)),
            pl.BlockSpec((256, 1), lambda i: (0, 0)),
            pl.BlockSpec((1, 1), lambda i: (0, 0)),
            pl.BlockSpec((BN, 1), lambda i: (i, 0)),
        ],
        out_specs=pl.BlockSpec((16, 1), lambda i: (0, 0)),
        out_shape=jax.ShapeDtypeStruct((16, 1), jnp.float32),
    )(x2, W_out, b_out, batch_col)


def kernel(node_feat, node_opcode, node_config_feat, node_config_ids,
           edge_index, batch, W_feat, b_feat, emb_table, W_in, b_in, convs,
           W_out, b_out):
    nf_pad = jnp.pad(node_feat, ((0, N_PAD - N_RAW), (0, 0)))
    b_feat2 = b_feat[None, :]
    Wi1 = W_in[:20]
    Wi2 = jnp.pad(W_in[20:32], ((0, 4), (0, 0)))
    Wi3 = jnp.pad(W_in[32:50], ((0, 14), (0, 0)))
    b_in2 = b_in[None, :]
    ids32 = node_config_ids.astype(jnp.int32)
    ids_f = jnp.pad(ids32.astype(jnp.float32), (0, NCFG_PAD - 1000),
                    constant_values=-1.0)[None, :]
    ids_i = jnp.pad(ids32, (0, NCFG_PAD - 1000), constant_values=-1)
    cfg32 = jnp.pad(node_config_feat, ((0, NCFG_PAD - 1000), (0, 32 - 18)))
    opc2 = jnp.pad(node_opcode.astype(jnp.int32),
                   (0, N_PAD - N_RAW)).reshape(16, 5, 128)
    embp = jnp.pad(emb_table, ((0, 8), (0, 4)))

    src = edge_index[0].astype(jnp.int32)
    dst = edge_index[1].astype(jnp.int32)
    src_pad = jnp.pad(src, (0, E_PAD - E_RAW))
    dstA = jnp.pad(dst, (0, E_PAD - E_RAW),
                   constant_values=N_RAW).reshape(EROWS, 128)
    srcq = [(src_pad + q * N_PAD).reshape(EROWS, 128) for q in range(4)]

    cntparts, op16, cfg_eff = _sc_pre(dstA, opc2, embp, ids_i, cfg32)

    x2, inv3 = _tc_pre(nf_pad, op16, cfg_eff, ids_f, cntparts, W_feat,
                       b_feat2, Wi1, Wi2, Wi3, b_in2)
    invf = inv3.reshape(N_PAD)

    ci = 32
    for p in convs:
        co = p["Wl"].shape[1]
        ki = x2.shape[0]
        ko = 4 if co == 256 else 2
        ch = ci // ki
        xflat = x2.reshape(ki * N_PAD, ch)
        parts = []
        for half in range(ki // 2):
            mf = _sc_spmv(ch)(xflat, srcq[2 * half], srcq[2 * half + 1],
                              dstA, invf)
            parts.append(mf.reshape(2, N_PAD, ch))
        mean2 = jnp.concatenate(parts, axis=0) if len(parts) > 1 else parts[0]
        x2 = _tc_layer(ci, co, ki, ko)(x2, mean2, p["Wl"], p["Wr"],
                                       p["bl"][None, :])
        ci = co

    batch_col = jnp.pad(batch.astype(jnp.float32), (0, N_PAD - N_RAW),
                        constant_values=16.0)[:, None]
    out = _tc_final(x2, W_out, b_out[None, :], batch_col)
    return out[:, 0]


# speedup vs baseline: 1.0300x; 1.0300x over previous
import functools

import jax
import jax.numpy as jnp
from jax import lax
from jax.experimental import pallas as pl
from jax.experimental.pallas import tpu as pltpu
from jax.experimental.pallas import tpu_sc as plsc

N_RAW = 10000
N_PAD = 10240
BN = 1280
NB = N_PAD // BN
NCFG_PAD = 1024
E_RAW = 320000
E_PAD = 327680
EROWS = 2560
TROWS = EROWS // 16

_SC_MESH = plsc.VectorSubcoreMesh(core_axis_name="c", subcore_axis_name="s",
                                  num_cores=2, num_subcores=16)
_SC_PARAMS = pltpu.CompilerParams(needs_layout_passes=False,
                                  use_tc_tiling_on_sc=False)


def _i16(val):
    return jnp.zeros((16,), jnp.int32) + val


def _sc_pre_body(dst_ref, opc_ref, emb_ref, ids_ref, cfg_ref,
                 cnt_out, op16_out, cfg_out,
                 dstv, cntw, opcv, oprows, idsv, cfgv, sem):
    c = lax.axis_index("c")
    s = lax.axis_index("s")

    @pl.when(c == 0)
    def _():
        zero16 = jnp.zeros((16,), jnp.float32)

        def zrow(i, carry):
            cntw[pl.ds(i * 16, 16)] = zero16
            return carry
        lax.fori_loop(0, N_PAD // 16, zrow, 0)

        pltpu.sync_copy(dst_ref.at[pl.ds(s * TROWS, TROWS)], dstv)
        ones = jnp.zeros((16,), jnp.float32) + 1.0

        def count(i, carry):
            r = i // 8
            k = i % 8
            d16 = dstv[r, pl.ds(k * 16, 16)]
            plsc.addupdate_scatter(cntw, [d16], ones)
            return carry
        lax.fori_loop(0, TROWS * 8, count, 0)
        pltpu.sync_copy(cntw, cnt_out.at[s])

    @pl.when(c == 1)
    def _():
        pltpu.sync_copy(opc_ref.at[s], opcv)
        for j in range(5):
            pltpu.async_copy(emb_ref.at[opcv.at[j]], oprows, sem).wait()
            pltpu.sync_copy(oprows, op16_out.at[pl.ds(s * 640 + j * 128, 128)])

        pltpu.sync_copy(ids_ref, idsv)
        pltpu.sync_copy(cfg_ref.at[pl.ds(s * 64, 64)], cfgv)
        iota16 = jnp.arange(16, dtype=jnp.int32)

        def cbody(j, carry):
            cc = s * 64 + j
            myid = plsc.load_gather(idsv, [_i16(cc)])

            def chunk(k, acc):
                vals = idsv[pl.ds(k * 16, 16)]
                pos = iota16 + k * 16
                hit = jnp.where((vals == myid) & (pos > cc), 1.0, 0.0)
                return jnp.maximum(acc, hit)
            acc = lax.fori_loop(0, 64, chunk, jnp.zeros((16,), jnp.float32))
            keep = jnp.where(jnp.max(acc) > 0.0, 0.0, 1.0)
            cfgv[j, pl.ds(0, 16)] = cfgv[j, pl.ds(0, 16)] * keep
            cfgv[j, pl.ds(16, 16)] = cfgv[j, pl.ds(16, 16)] * keep
            return carry
        lax.fori_loop(0, 64, cbody, 0)
        pltpu.sync_copy(cfgv, cfg_out.at[pl.ds(s * 64, 64)])


def _sc_pre(dstA, opc2, embp, ids_i, cfg32):
    f32 = jnp.float32
    return pl.kernel(
        _sc_pre_body,
        out_type=[
            jax.ShapeDtypeStruct((16, N_PAD), f32),
            jax.ShapeDtypeStruct((N_PAD, 16), f32),
            jax.ShapeDtypeStruct((NCFG_PAD, 32), f32)
        ],
        mesh=_SC_MESH,
        scratch_types=[
            pltpu.VMEM((TROWS, 128), jnp.int32),
            pltpu.VMEM((N_PAD,), f32),
            pltpu.VMEM((5, 128), jnp.int32),
            pltpu.VMEM((128, 16), f32),
            pltpu.VMEM((NCFG_PAD,), jnp.int32),
            pltpu.VMEM((64, 32), f32),
            pltpu.SemaphoreType.DMA,
        ],
        compiler_params=_SC_PARAMS,
    )(dstA, opc2, embp, ids_i, cfg32)


def _spmv_body(x_ref, src0_ref, src1_ref, dst_ref, inv_ref, out_ref,
               sidx, didx, wb, invv, acc_sh, gsem, ssem, *bufs, ch, nbuf, w):
    c = lax.axis_index("c")
    s = lax.axis_index("s")
    zero16 = jnp.zeros((16,), jnp.float32)

    def zrow(i, carry):
        for m in range(ch // 16):
            wb[i, pl.ds(m * 16, 16)] = zero16
        return carry
    lax.fori_loop(0, 64, zrow, 0)

    def zslab(t, carry):
        pltpu.sync_copy(wb, acc_sh.at[pl.ds(s * 640 + t * 64, 64)])
        return carry
    lax.fori_loop(0, 10, zslab, 0)
    plsc.subcore_barrier()

    trows = 20480 // w
    @pl.when(c == 0)
    def _():
        pltpu.sync_copy(src0_ref.at[pl.ds(s * trows, trows)], sidx)

    @pl.when(c == 1)
    def _():
        pltpu.sync_copy(src1_ref.at[pl.ds(s * trows, trows)], sidx)
    pltpu.sync_copy(dst_ref.at[pl.ds(s * trows, trows)], didx)

    dep = nbuf // 2

    def fire_g(b, t):
        pltpu.async_copy(x_ref.at[sidx.at[t]], bufs[b], gsem.at[b])

    def fire_s(b, u):
        pltpu.async_copy(bufs[b], acc_sh.at[didx.at[u]], ssem.at[b], add=True)

    def wait_g(b):
        pltpu.make_async_copy(x_ref.at[sidx.at[0]], bufs[b], gsem.at[b]).wait()

    def wait_s(b):
        pltpu.make_async_copy(bufs[b], acc_sh.at[didx.at[0]],
                              ssem.at[b]).wait()

    for j in range(16):
        b = j % nbuf
        if j >= nbuf:
            wait_s(b)
        fire_g(b, j)
        if j >= dep:
            wait_g((j - dep) % nbuf)
            fire_s((j - dep) % nbuf, j - dep)

    def sup(q, carry):
        t0 = q * 16
        for j in range(16):
            b = j % nbuf
            wait_s(b)
            fire_g(b, t0 + j)
            wait_g((j - dep) % nbuf)
            fire_s((j - dep) % nbuf, t0 + j - dep)
        return carry
    lax.fori_loop(1, trows // 16, sup, 0)

    for u in range(trows - dep, trows):
        wait_g(u % nbuf)
        fire_s(u % nbuf, u)
    for b in range(nbuf):
        wait_s(b)
    plsc.subcore_barrier()

    pltpu.sync_copy(inv_ref.at[pl.ds(s * 640, 640)], invv)

    def wo(t, carry):
        r0 = s * 640 + t * 64
        pltpu.sync_copy(acc_sh.at[pl.ds(r0, 64)], wb)

        def row(r, carry2):
            iv = plsc.load_gather(invv, [_i16(t * 64 + r)])
            for m in range(ch // 16):
                wb[r, pl.ds(m * 16, 16)] = wb[r, pl.ds(m * 16, 16)] * iv
            return carry2
        lax.fori_loop(0, 64, row, 0)
        pltpu.sync_copy(wb, out_ref.at[pl.ds(c * N_PAD + r0, 64)])
        return carry
    lax.fori_loop(0, 10, wo, 0)


@functools.lru_cache(maxsize=None)
def _sc_spmv(ch):
    f32 = jnp.float32
    nbuf = 8
    w = 64 if ch == 64 else 128
    trows = 20480 // w
    return pl.kernel(
        functools.partial(_spmv_body, ch=ch, nbuf=nbuf, w=w),
        out_type=jax.ShapeDtypeStruct((2 * N_PAD, ch), f32),
        mesh=_SC_MESH,
        scratch_types=[
            pltpu.VMEM((trows, w), jnp.int32),
            pltpu.VMEM((trows, w), jnp.int32),
            pltpu.VMEM((64, ch), f32),
            pltpu.VMEM((640,), f32),
            pltpu.VMEM_SHARED((N_PAD, ch), f32),
            pltpu.SemaphoreType.DMA((nbuf,)),
            pltpu.SemaphoreType.DMA((nbuf,)),
        ] + [pltpu.VMEM((w, ch), f32)] * nbuf,
        compiler_params=_SC_PARAMS,
    )


def _pre_body(nf_ref, op16_ref, cfg_eff_ref, ids_ref, cnt_ref, wf_ref, bf_ref,
              wi1_ref, wi2_ref, wi3_ref, bi_ref, out_ref, inv_ref):
    cnt = jnp.sum(cnt_ref[...], axis=0, keepdims=True)
    inv_ref[0] = 1.0 / jnp.maximum(cnt, 1.0)
    feat = jnp.log1p(jax.nn.relu(nf_ref[...]))
    femb = jnp.dot(feat, wf_ref[...], preferred_element_type=jnp.float32) + bf_ref[...]
    t1 = jnp.dot(femb, wi1_ref[...], preferred_element_type=jnp.float32)
    t2 = jnp.dot(op16_ref[...], wi2_ref[...], preferred_element_type=jnp.float32)
    u = jnp.dot(cfg_eff_ref[...], wi3_ref[...], preferred_element_type=jnp.float32)
    ni = (pl.program_id(0) * BN
          + lax.broadcasted_iota(jnp.int32, (BN, 1), 0)).astype(jnp.float32)
    oh = (ni == ids_ref[...]).astype(jnp.float32)
    inj = jnp.dot(oh, u, preferred_element_type=jnp.float32)
    y = jax.nn.relu(t1 + t2 + inj + bi_ref[...])
    out_ref[0] = y[:, :16]
    out_ref[1] = y[:, 16:]


def _tc_pre(nf_pad, op16, cfg_eff, ids_f, cntparts, W_feat, b_feat,
            Wi1, Wi2, Wi3, b_in):
    return pl.pallas_call(
        _pre_body,
        grid=(NB,),
        in_specs=[
            pl.BlockSpec((BN, 140), lambda i: (i, 0)),
            pl.BlockSpec((BN, 16), lambda i: (i, 0)),
            pl.BlockSpec((NCFG_PAD, 32), lambda i: (0, 0)),
            pl.BlockSpec((1, NCFG_PAD), lambda i: (0, 0)),
            pl.BlockSpec((16, BN), lambda i: (0, i)),
            pl.BlockSpec((140, 20), lambda i: (0, 0)),
            pl.BlockSpec((1, 20), lambda i: (0, 0)),
            pl.BlockSpec((20, 32), lambda i: (0, 0)),
            pl.BlockSpec((16, 32), lambda i: (0, 0)),
            pl.BlockSpec((32, 32), lambda i: (0, 0)),
            pl.BlockSpec((1, 32), lambda i: (0, 0)),
        ],
        out_specs=[
            pl.BlockSpec((2, BN, 16), lambda i: (0, i, 0)),
            pl.BlockSpec((1, 1, BN), lambda i: (i, 0, 0)),
        ],
        out_shape=[
            jax.ShapeDtypeStruct((2, N_PAD, 16), jnp.float32),
            jax.ShapeDtypeStruct((NB, 1, BN), jnp.float32),
        ],
    )(nf_pad, op16, cfg_eff, ids_f, cntparts, W_feat, b_feat,
      Wi1, Wi2, Wi3, b_in)


def _layer_body(x_ref, m_ref, wl_ref, wr_ref, bl_ref, out_ref, *, ki, ko, co):
    xc = jnp.concatenate([x_ref[q] for q in range(ki)], axis=-1)
    mc = jnp.concatenate([m_ref[q] for q in range(ki)], axis=-1)
    y = jnp.dot(mc, wl_ref[...], preferred_element_type=jnp.float32)
    y = y + jnp.dot(xc, wr_ref[...], preferred_element_type=jnp.float32)
    y = jax.nn.relu(y + bl_ref[...])
    cq = co // ko
    for q in range(ko):
        out_ref[q] = y[:, q * cq:(q + 1) * cq]


@functools.lru_cache(maxsize=None)
def _tc_layer(ci, co, ki, ko):
    ciq, coq = ci // ki, co // ko
    return pl.pallas_call(
        functools.partial(_layer_body, ki=ki, ko=ko, co=co),
        grid=(NB,),
        in_specs=[
            pl.BlockSpec((ki, BN, ciq), lambda i: (0, i, 0)),
            pl.BlockSpec((ki, BN, ciq), lambda i: (0, i, 0)),
            pl.BlockSpec((ci, co), lambda i: (0, 0)),
            pl.BlockSpec((ci, co), lambda i: (0, 0)),
            pl.BlockSpec((1, co), lambda i: (0, 0)),
        ],
        out_specs=pl.BlockSpec((ko, BN, coq), lambda i: (0, i, 0)),
        out_shape=jax.ShapeDtypeStruct((ko, N_PAD, coq), jnp.float32),
    )


def _final_body(x_ref, wout_ref, bout_ref, batch_ref, out_ref):
    xc = jnp.concatenate([x_ref[q] for q in range(4)], axis=-1)
    v = jnp.dot(xc, wout_ref[...], preferred_element_type=jnp.float32) + bout_ref[...]
    gr = lax.broadcasted_iota(jnp.int32, (1, 16), 1).astype(jnp.float32)
    oh = (batch_ref[...] == gr).astype(jnp.float32)
    po = 0.001 * lax.dot_general(oh, v, (((0,), (0,)), ((), ())),
                                 preferred_element_type=jnp.float32)

    @pl.when(pl.program_id(0) == 0)
    def _():
        out_ref[...] = po

    @pl.when(pl.program_id(0) != 0)
    def _():
        out_ref[...] += po


def _tc_final(x2, W_out, b_out, batch_col):
    return pl.pallas_call(
        _final_body,
        grid=(NB,),
        in_specs=[
            pl.BlockSpec((4, BN, 64), lambda i: (0, i, 0)),
            pl.BlockSpec((256, 1), lambda i: (0, 0)),
            pl.BlockSpec((1, 1), lambda i: (0, 0)),
            pl.BlockSpec((BN, 1), lambda i: (i, 0)),
        ],
        out_specs=pl.BlockSpec((16, 1), lambda i: (0, 0)),
        out_shape=jax.ShapeDtypeStruct((16, 1), jnp.float32),
    )(x2, W_out, b_out, batch_col)


def kernel(node_feat, node_opcode, node_config_feat, node_config_ids,
           edge_index, batch, W_feat, b_feat, emb_table, W_in, b_in, convs,
           W_out, b_out):
    nf_pad = jnp.pad(node_feat, ((0, N_PAD - N_RAW), (0, 0)))
    b_feat2 = b_feat[None, :]
    Wi1 = W_in[:20]
    Wi2 = jnp.pad(W_in[20:32], ((0, 4), (0, 0)))
    Wi3 = jnp.pad(W_in[32:50], ((0, 14), (0, 0)))
    b_in2 = b_in[None, :]
    ids32 = node_config_ids.astype(jnp.int32)
    ids_f = jnp.pad(ids32.astype(jnp.float32), (0, NCFG_PAD - 1000),
                    constant_values=-1.0)[None, :]
    ids_i = jnp.pad(ids32, (0, NCFG_PAD - 1000), constant_values=-1)
    cfg32 = jnp.pad(node_config_feat, ((0, NCFG_PAD - 1000), (0, 32 - 18)))
    opc2 = jnp.pad(node_opcode.astype(jnp.int32),
                   (0, N_PAD - N_RAW)).reshape(16, 5, 128)
    embp = jnp.pad(emb_table, ((0, 8), (0, 4)))

    src = edge_index[0].astype(jnp.int32)
    dst = edge_index[1].astype(jnp.int32)
    src_pad = jnp.pad(src, (0, E_PAD - E_RAW))
    dst_pad = jnp.pad(dst, (0, E_PAD - E_RAW), constant_values=N_RAW)
    srcq = [src_pad + q * N_PAD for q in range(4)]

    cntparts, op16, cfg_eff = _sc_pre(dst_pad.reshape(EROWS, 128), opc2,
                                      embp, ids_i, cfg32)

    x2, inv3 = _tc_pre(nf_pad, op16, cfg_eff, ids_f, cntparts, W_feat,
                       b_feat2, Wi1, Wi2, Wi3, b_in2)
    invf = inv3.reshape(N_PAD)

    ci = 32
    for p in convs:
        co = p["Wl"].shape[1]
        ki = x2.shape[0]
        ko = 4 if co == 256 else 2
        ch = ci // ki
        xflat = x2.reshape(ki * N_PAD, ch)
        w = 64 if ch == 64 else 128
        dstA = dst_pad.reshape(E_PAD // w, w)
        parts = []
        for half in range(ki // 2):
            mf = _sc_spmv(ch)(xflat, srcq[2 * half].reshape(E_PAD // w, w),
                              srcq[2 * half + 1].reshape(E_PAD // w, w),
                              dstA, invf)
            parts.append(mf.reshape(2, N_PAD, ch))
        mean2 = jnp.concatenate(parts, axis=0) if len(parts) > 1 else parts[0]
        x2 = _tc_layer(ci, co, ki, ko)(x2, mean2, p["Wl"], p["Wr"],
                                       p["bl"][None, :])
        ci = co

    batch_col = jnp.pad(batch.astype(jnp.float32), (0, N_PAD - N_RAW),
                        constant_values=16.0)[:, None]
    out = _tc_final(x2, W_out, b_out[None, :], batch_col)
    return out[:, 0]


# speedup vs baseline: 1.0591x; 1.0282x over previous
import functools

import jax
import jax.numpy as jnp
from jax import lax
from jax.experimental import pallas as pl
from jax.experimental.pallas import tpu as pltpu
from jax.experimental.pallas import tpu_sc as plsc

N_RAW = 10000
N_PAD = 10240
BN = 1280
NB = N_PAD // BN
NCFG_PAD = 1024
E_RAW = 320000
E_PAD = 327680
EROWS = 2560
TROWS = EROWS // 16

_SC_MESH = plsc.VectorSubcoreMesh(core_axis_name="c", subcore_axis_name="s",
                                  num_cores=2, num_subcores=16)
_SC_PARAMS = pltpu.CompilerParams(needs_layout_passes=False,
                                  use_tc_tiling_on_sc=False)


def _i16(val):
    return jnp.zeros((16,), jnp.int32) + val


def _sc_pre_body(dst_ref, opc_ref, emb_ref, ids_ref, cfg_ref,
                 cnt_out, op16_out, cfg_out,
                 dstv, cntw, opcv, oprows, idsv, cfgv, sem):
    c = lax.axis_index("c")
    s = lax.axis_index("s")

    @pl.when(c == 0)
    def _():
        zero16 = jnp.zeros((16,), jnp.float32)

        def zrow(i, carry):
            cntw[pl.ds(i * 16, 16)] = zero16
            return carry
        lax.fori_loop(0, N_PAD // 16, zrow, 0)

        pltpu.sync_copy(dst_ref.at[pl.ds(s * TROWS, TROWS)], dstv)
        ones = jnp.zeros((16,), jnp.float32) + 1.0

        def count(i, carry):
            r = i // 8
            k = i % 8
            d16 = dstv[r, pl.ds(k * 16, 16)]
            plsc.addupdate_scatter(cntw, [d16], ones)
            return carry
        lax.fori_loop(0, TROWS * 8, count, 0)
        pltpu.sync_copy(cntw, cnt_out.at[s])

    @pl.when(c == 1)
    def _():
        pltpu.sync_copy(opc_ref.at[s], opcv)
        for j in range(5):
            pltpu.async_copy(emb_ref.at[opcv.at[j]], oprows, sem).wait()
            pltpu.sync_copy(oprows, op16_out.at[pl.ds(s * 640 + j * 128, 128)])

        pltpu.sync_copy(ids_ref, idsv)
        pltpu.sync_copy(cfg_ref.at[pl.ds(s * 64, 64)], cfgv)
        iota16 = jnp.arange(16, dtype=jnp.int32)

        def cbody(j, carry):
            cc = s * 64 + j
            myid = plsc.load_gather(idsv, [_i16(cc)])

            def chunk(k, acc):
                vals = idsv[pl.ds(k * 16, 16)]
                pos = iota16 + k * 16
                hit = jnp.where((vals == myid) & (pos > cc), 1.0, 0.0)
                return jnp.maximum(acc, hit)
            acc = lax.fori_loop(0, 64, chunk, jnp.zeros((16,), jnp.float32))
            keep = jnp.where(jnp.max(acc) > 0.0, 0.0, 1.0)
            cfgv[j, pl.ds(0, 16)] = cfgv[j, pl.ds(0, 16)] * keep
            cfgv[j, pl.ds(16, 16)] = cfgv[j, pl.ds(16, 16)] * keep
            return carry
        lax.fori_loop(0, 64, cbody, 0)
        pltpu.sync_copy(cfgv, cfg_out.at[pl.ds(s * 64, 64)])


def _sc_pre(dstA, opc2, embp, ids_i, cfg32):
    f32 = jnp.float32
    return pl.kernel(
        _sc_pre_body,
        out_type=[
            jax.ShapeDtypeStruct((16, N_PAD), f32),
            jax.ShapeDtypeStruct((N_PAD, 16), f32),
            jax.ShapeDtypeStruct((NCFG_PAD, 32), f32)
        ],
        mesh=_SC_MESH,
        scratch_types=[
            pltpu.VMEM((TROWS, 128), jnp.int32),
            pltpu.VMEM((N_PAD,), f32),
            pltpu.VMEM((5, 128), jnp.int32),
            pltpu.VMEM((128, 16), f32),
            pltpu.VMEM((NCFG_PAD,), jnp.int32),
            pltpu.VMEM((64, 32), f32),
            pltpu.SemaphoreType.DMA,
        ],
        compiler_params=_SC_PARAMS,
    )(dstA, opc2, embp, ids_i, cfg32)


def _spmv_body(x_ref, src0_ref, src1_ref, dst_ref, out_ref,
               sidx, didx, wb, acc_sh, gsem, ssem, *bufs, ch, nbuf, w):
    c = lax.axis_index("c")
    s = lax.axis_index("s")
    zero16 = jnp.zeros((16,), jnp.float32)

    def zrow(i, carry):
        for m in range(ch // 16):
            wb[i, pl.ds(m * 16, 16)] = zero16
        return carry
    lax.fori_loop(0, 64, zrow, 0)

    def zslab(t, carry):
        pltpu.sync_copy(wb, acc_sh.at[pl.ds(s * 640 + t * 64, 64)])
        return carry
    lax.fori_loop(0, 10, zslab, 0)
    plsc.subcore_barrier()

    trows = 20480 // w
    @pl.when(c == 0)
    def _():
        pltpu.sync_copy(src0_ref.at[pl.ds(s * trows, trows)], sidx)

    @pl.when(c == 1)
    def _():
        pltpu.sync_copy(src1_ref.at[pl.ds(s * trows, trows)], sidx)
    pltpu.sync_copy(dst_ref.at[pl.ds(s * trows, trows)], didx)

    dep = nbuf // 2

    def fire_g(b, t):
        pltpu.async_copy(x_ref.at[sidx.at[t]], bufs[b], gsem.at[b])

    def fire_s(b, u):
        pltpu.async_copy(bufs[b], acc_sh.at[didx.at[u]], ssem.at[b], add=True)

    def wait_g(b):
        pltpu.make_async_copy(x_ref.at[sidx.at[0]], bufs[b], gsem.at[b]).wait()

    def wait_s(b):
        pltpu.make_async_copy(bufs[b], acc_sh.at[didx.at[0]],
                              ssem.at[b]).wait()

    for j in range(16):
        b = j % nbuf
        if j >= nbuf:
            wait_s(b)
        fire_g(b, j)
        if j >= dep:
            wait_g((j - dep) % nbuf)
            fire_s((j - dep) % nbuf, j - dep)

    def sup(q, carry):
        t0 = q * 16
        for j in range(16):
            b = j % nbuf
            wait_s(b)
            fire_g(b, t0 + j)
            wait_g((j - dep) % nbuf)
            fire_s((j - dep) % nbuf, t0 + j - dep)
        return carry
    lax.fori_loop(1, trows // 16, sup, 0)

    for u in range(trows - dep, trows):
        wait_g(u % nbuf)
        fire_s(u % nbuf, u)
    for b in range(nbuf):
        wait_s(b)
    plsc.subcore_barrier()

    pltpu.sync_copy(acc_sh.at[pl.ds(s * 640, 640)],
                    out_ref.at[pl.ds(c * N_PAD + s * 640, 640)])


@functools.lru_cache(maxsize=None)
def _sc_spmv(ch):
    f32 = jnp.float32
    nbuf = 8
    w = 64 if ch == 64 else 128
    trows = 20480 // w
    return pl.kernel(
        functools.partial(_spmv_body, ch=ch, nbuf=nbuf, w=w),
        out_type=jax.ShapeDtypeStruct((2 * N_PAD, ch), f32),
        mesh=_SC_MESH,
        scratch_types=[
            pltpu.VMEM((trows, w), jnp.int32),
            pltpu.VMEM((trows, w), jnp.int32),
            pltpu.VMEM((64, ch), f32),
            pltpu.VMEM_SHARED((N_PAD, ch), f32),
            pltpu.SemaphoreType.DMA((nbuf,)),
            pltpu.SemaphoreType.DMA((nbuf,)),
        ] + [pltpu.VMEM((w, ch), f32)] * nbuf,
        compiler_params=_SC_PARAMS,
    )


def _pre_body(nf_ref, op16_ref, cfg_eff_ref, ids_ref, cnt_ref, wf_ref, bf_ref,
              wi1_ref, wi2_ref, wi3_ref, bi_ref, out_ref, inv_ref):
    cnt = jnp.sum(cnt_ref[...], axis=0, keepdims=True)
    inv_ref[...] = 1.0 / jnp.maximum(cnt, 1.0)
    feat = jnp.log1p(jax.nn.relu(nf_ref[...]))
    femb = jnp.dot(feat, wf_ref[...], preferred_element_type=jnp.float32) + bf_ref[...]
    t1 = jnp.dot(femb, wi1_ref[...], preferred_element_type=jnp.float32)
    t2 = jnp.dot(op16_ref[...], wi2_ref[...], preferred_element_type=jnp.float32)
    u = jnp.dot(cfg_eff_ref[...], wi3_ref[...], preferred_element_type=jnp.float32)
    ni = (pl.program_id(0) * BN
          + lax.broadcasted_iota(jnp.int32, (BN, 1), 0)).astype(jnp.float32)
    oh = (ni == ids_ref[...]).astype(jnp.float32)
    inj = jnp.dot(oh, u, preferred_element_type=jnp.float32)
    y = jax.nn.relu(t1 + t2 + inj + bi_ref[...])
    out_ref[0] = y[:, :16]
    out_ref[1] = y[:, 16:]


def _tc_pre(nf_pad, op16, cfg_eff, ids_f, cntparts, W_feat, b_feat,
            Wi1, Wi2, Wi3, b_in):
    return pl.pallas_call(
        _pre_body,
        grid=(NB,),
        in_specs=[
            pl.BlockSpec((BN, 140), lambda i: (i, 0)),
            pl.BlockSpec((BN, 16), lambda i: (i, 0)),
            pl.BlockSpec((NCFG_PAD, 32), lambda i: (0, 0)),
            pl.BlockSpec((1, NCFG_PAD), lambda i: (0, 0)),
            pl.BlockSpec((16, BN), lambda i: (0, i)),
            pl.BlockSpec((140, 20), lambda i: (0, 0)),
            pl.BlockSpec((1, 20), lambda i: (0, 0)),
            pl.BlockSpec((20, 32), lambda i: (0, 0)),
            pl.BlockSpec((16, 32), lambda i: (0, 0)),
            pl.BlockSpec((32, 32), lambda i: (0, 0)),
            pl.BlockSpec((1, 32), lambda i: (0, 0)),
        ],
        out_specs=[
            pl.BlockSpec((2, BN, 16), lambda i: (0, i, 0)),
            pl.BlockSpec((1, BN), lambda i: (0, i)),
        ],
        out_shape=[
            jax.ShapeDtypeStruct((2, N_PAD, 16), jnp.float32),
            jax.ShapeDtypeStruct((1, N_PAD), jnp.float32),
        ],
    )(nf_pad, op16, cfg_eff, ids_f, cntparts, W_feat, b_feat,
      Wi1, Wi2, Wi3, b_in)


def _layer_body(x_ref, m_ref, inv_ref, wl_ref, wr_ref, bl_ref, out_ref,
                *, ki, ko, co):
    xc = jnp.concatenate([x_ref[q] for q in range(ki)], axis=-1)
    mc = jnp.concatenate([m_ref[q] for q in range(ki)], axis=-1)
    mc = mc * inv_ref[...]
    y = jnp.dot(mc, wl_ref[...], preferred_element_type=jnp.float32)
    y = y + jnp.dot(xc, wr_ref[...], preferred_element_type=jnp.float32)
    y = jax.nn.relu(y + bl_ref[...])
    cq = co // ko
    for q in range(ko):
        out_ref[q] = y[:, q * cq:(q + 1) * cq]


@functools.lru_cache(maxsize=None)
def _tc_layer(ci, co, ki, ko):
    ciq, coq = ci // ki, co // ko
    return pl.pallas_call(
        functools.partial(_layer_body, ki=ki, ko=ko, co=co),
        grid=(NB,),
        in_specs=[
            pl.BlockSpec((ki, BN, ciq), lambda i: (0, i, 0)),
            pl.BlockSpec((ki, BN, ciq), lambda i: (0, i, 0)),
            pl.BlockSpec((BN, 1), lambda i: (i, 0)),
            pl.BlockSpec((ci, co), lambda i: (0, 0)),
            pl.BlockSpec((ci, co), lambda i: (0, 0)),
            pl.BlockSpec((1, co), lambda i: (0, 0)),
        ],
        out_specs=pl.BlockSpec((ko, BN, coq), lambda i: (0, i, 0)),
        out_shape=jax.ShapeDtypeStruct((ko, N_PAD, coq), jnp.float32),
    )


def _final_body(x_ref, wout_ref, bout_ref, batch_ref, out_ref):
    xc = jnp.concatenate([x_ref[q] for q in range(4)], axis=-1)
    v = jnp.dot(xc, wout_ref[...], preferred_element_type=jnp.float32) + bout_ref[...]
    gr = lax.broadcasted_iota(jnp.int32, (1, 16), 1).astype(jnp.float32)
    oh = (batch_ref[...] == gr).astype(jnp.float32)
    po = 0.001 * lax.dot_general(oh, v, (((0,), (0,)), ((), ())),
                                 preferred_element_type=jnp.float32)

    @pl.when(pl.program_id(0) == 0)
    def _():
        out_ref[...] = po

    @pl.when(pl.program_id(0) != 0)
    def _():
        out_ref[...] += po


def _tc_final(x2, W_out, b_out, batch_col):
    return pl.pallas_call(
        _final_body,
        grid=(NB,),
        in_specs=[
            pl.BlockSpec((4, BN, 64), lambda i: (0, i, 0)),
            pl.BlockSpec((256, 1), lambda i: (0, 0)),
            pl.BlockSpec((1, 1), lambda i: (0, 0)),
            pl.BlockSpec((BN, 1), lambda i: (i, 0)),
        ],
        out_specs=pl.BlockSpec((16, 1), lambda i: (0, 0)),
        out_shape=jax.ShapeDtypeStruct((16, 1), jnp.float32),
    )(x2, W_out, b_out, batch_col)


def kernel(node_feat, node_opcode, node_config_feat, node_config_ids,
           edge_index, batch, W_feat, b_feat, emb_table, W_in, b_in, convs,
           W_out, b_out):
    nf_pad = jnp.pad(node_feat, ((0, N_PAD - N_RAW), (0, 0)))
    b_feat2 = b_feat[None, :]
    Wi1 = W_in[:20]
    Wi2 = jnp.pad(W_in[20:32], ((0, 4), (0, 0)))
    Wi3 = jnp.pad(W_in[32:50], ((0, 14), (0, 0)))
    b_in2 = b_in[None, :]
    ids32 = node_config_ids.astype(jnp.int32)
    ids_f = jnp.pad(ids32.astype(jnp.float32), (0, NCFG_PAD - 1000),
                    constant_values=-1.0)[None, :]
    ids_i = jnp.pad(ids32, (0, NCFG_PAD - 1000), constant_values=-1)
    cfg32 = jnp.pad(node_config_feat, ((0, NCFG_PAD - 1000), (0, 32 - 18)))
    opc2 = jnp.pad(node_opcode.astype(jnp.int32),
                   (0, N_PAD - N_RAW)).reshape(16, 5, 128)
    embp = jnp.pad(emb_table, ((0, 8), (0, 4)))

    src = edge_index[0].astype(jnp.int32)
    dst = edge_index[1].astype(jnp.int32)
    src_pad = jnp.pad(src, (0, E_PAD - E_RAW))
    dst_pad = jnp.pad(dst, (0, E_PAD - E_RAW), constant_values=N_RAW)
    srcq = [src_pad + q * N_PAD for q in range(4)]

    cntparts, op16, cfg_eff = _sc_pre(dst_pad.reshape(EROWS, 128), opc2,
                                      embp, ids_i, cfg32)

    x2, invrow = _tc_pre(nf_pad, op16, cfg_eff, ids_f, cntparts, W_feat,
                         b_feat2, Wi1, Wi2, Wi3, b_in2)
    invcol = invrow.reshape(N_PAD, 1)

    ci = 32
    for p in convs:
        co = p["Wl"].shape[1]
        ki = x2.shape[0]
        ko = 4 if co == 256 else 2
        ch = ci // ki
        xflat = x2.reshape(ki * N_PAD, ch)
        w = 64 if ch == 64 else 128
        dstA = dst_pad.reshape(E_PAD // w, w)
        parts = []
        for half in range(ki // 2):
            mf = _sc_spmv(ch)(xflat, srcq[2 * half].reshape(E_PAD // w, w),
                              srcq[2 * half + 1].reshape(E_PAD // w, w),
                              dstA)
            parts.append(mf.reshape(2, N_PAD, ch))
        mean2 = jnp.concatenate(parts, axis=0) if len(parts) > 1 else parts[0]
        x2 = _tc_layer(ci, co, ki, ko)(x2, mean2, invcol, p["Wl"], p["Wr"],
                                       p["bl"][None, :])
        ci = co

    batch_col = jnp.pad(batch.astype(jnp.float32), (0, N_PAD - N_RAW),
                        constant_values=16.0)[:, None]
    out = _tc_final(x2, W_out, b_out[None, :], batch_col)
    return out[:, 0]
